# baseline (speedup 1.0000x reference)
import functools

import jax
import jax.numpy as jnp
from jax import lax
from jax.experimental import pallas as pl
from jax.experimental.pallas import tpu as pltpu
from jax.experimental.pallas import tpu_sc as plsc

E = 8
TOPK = 2
D = 1024
H = 512
T = 2048
A = T * TOPK

B = 128
LOGB = 7
NB = A // B + E

NTILES = 32
CHUNK = A // NTILES
NV = CHUNK // 16


def _splat(vec, lanes, e):
    return jnp.zeros((16,), jnp.int32) + jnp.sum(jnp.where(lanes == e, vec, 0))


def _route_body(ifl_hbm, x_hbm, pos_hbm, gid_hbm, xs_hbm, ifl_v, pos_v, tok_v,
                gid_v, rows_a, rows_b, rows_c, sem_g, sem_s):
    c = lax.axis_index("c")
    s = lax.axis_index("s")
    wid = c * 16 + s
    lanes = jnp.arange(16, dtype=jnp.int32)
    bufs = [rows_a, rows_b, rows_c]

    for q in range(4):
        for h in range(2):
            r = wid * CHUNK + q * 32 + h * 16 + lanes
            tok_v[q, pl.ds(h * 16, 16)] = r >> 1
    gathers = []
    for q in range(3):
        gathers.append(
            pltpu.async_copy(x_hbm.at[tok_v.at[q]], bufs[q], sem_g)
        )

    pltpu.sync_copy(ifl_hbm, ifl_v)

    def step(v, carry):
        tot, pre = carry
        kv = ifl_v[pl.ds(v * 16, 16)]
        for e in range(E):
            m = kv == e
            cnt = jnp.sum(jnp.where(m, 1, 0))
            hit = jnp.where(lanes == e, cnt, 0)
            tot = tot + hit
            pre = pre + jnp.where(v < wid * NV, hit, jnp.zeros((16,), jnp.int32))
        return tot, pre

    zero16 = jnp.zeros((16,), jnp.int32)
    tot, pre = lax.fori_loop(0, A // 16, step, (zero16, zero16))

    blocks = (tot + (B - 1)) >> LOGB
    endblk = jnp.cumsum(blocks)
    startblk = endblk - blocks
    base = (startblk << LOGB) + pre

    @pl.when(wid == 0)
    def _():
        for j in range(3):
            bvec = lanes + 16 * j
            gid = jnp.zeros((16,), jnp.int32)
            for e in range(E):
                gid = gid + jnp.where(_splat(endblk, lanes, e) <= bvec, 1, 0)
            gid_v[pl.ds(j * 16, 16)] = jnp.minimum(gid, E - 1)
        pltpu.sync_copy(gid_v, gid_hbm)

    run = base
    for v in range(NV):
        kv = ifl_v[pl.ds((wid * NV + v) * 16, 16)]
        posv = jnp.zeros((16,), jnp.int32)
        for e in range(E):
            m = kv == e
            rank = jnp.cumsum(jnp.where(m, 1, 0)) - 1
            posv = jnp.where(m, _splat(run, lanes, e) + rank, posv)
            cnt = jnp.sum(jnp.where(m, 1, 0))
            run = run + jnp.where(lanes == e, cnt, 0)
        pos_v[v // 2, pl.ds((v % 2) * 16, 16)] = posv

    pltpu.sync_copy(pos_v, pos_hbm.at[wid])

    gathers[0].wait()
    s0 = pltpu.async_copy(bufs[0], xs_hbm.at[pos_v.at[0]], sem_s)
    gathers[1].wait()
    s1 = pltpu.async_copy(bufs[1], xs_hbm.at[pos_v.at[1]], sem_s)
    s0.wait()
    g3 = pltpu.async_copy(x_hbm.at[tok_v.at[3]], bufs[0], sem_g)
    gathers[2].wait()
    s2 = pltpu.async_copy(bufs[2], xs_hbm.at[pos_v.at[2]], sem_s)
    g3.wait()
    s3 = pltpu.async_copy(bufs[0], xs_hbm.at[pos_v.at[3]], sem_s)
    s1.wait()
    s2.wait()
    s3.wait()


def _group_body(gid_ref, x_ref, w1g_ref, w1u_ref, w2_ref, o_ref):
    xb = x_ref[...]
    gate = lax.dot_general(
        xb, w1g_ref[0], (((1,), (1,)), ((), ())),
        preferred_element_type=jnp.float32,
    )
    up = lax.dot_general(
        xb, w1u_ref[0], (((1,), (1,)), ((), ())),
        preferred_element_type=jnp.float32,
    )
    a = gate * lax.logistic(gate) * up
    o_ref[...] = lax.dot_general(
        a, w2_ref[0], (((1,), (1,)), ((), ())),
        preferred_element_type=jnp.float32,
    )


def _combine_body(os_hbm, pos_hbm, w_hbm, y_hbm, pos_v, w_v, r0, r1, r2, r3,
                  y_a, y_b, sem_g, sem_s):
    c = lax.axis_index("c")
    s = lax.axis_index("s")
    wid = c * 16 + s
    rows = [r0, r1, r2, r3]
    ys = [y_a, y_b]

    pltpu.sync_copy(pos_hbm.at[wid], pos_v)
    pltpu.sync_copy(w_hbm.at[pl.ds(wid * CHUNK, CHUNK)], w_v)

    def idx(q):
        return pos_v.at[q // 2, pl.ds((q % 2) * 16, 16)]

    gathers = [pltpu.async_copy(os_hbm.at[idx(q)], rows[q], sem_g)
               for q in range(4)]
    writes = [None, None]
    for q in range(8):
        gathers[q].wait()
        rv = rows[q % 4]
        yv = ys[q % 2]
        if writes[q % 2] is not None:
            writes[q % 2].wait()
        wv = w_v[pl.ds(q * 16, 16)]
        for j in range(8):
            w0 = wv[2 * j]
            w1 = wv[2 * j + 1]

            def inner(u, _, j=j, w0=w0, w1=w1, rv=rv, yv=yv):
                for p in range(8):
                    sl = pl.ds((u * 8 + p) * 16, 16)
                    yv[j, sl] = rv[2 * j, sl] * w0 + rv[2 * j + 1, sl] * w1
                return 0

            lax.fori_loop(0, D // 128, inner, 0)
        if q + 4 < 8:
            gathers.append(
                pltpu.async_copy(os_hbm.at[idx(q + 4)], rv, sem_g)
            )
        writes[q % 2] = pltpu.async_copy(
            yv, y_hbm.at[pl.ds(wid * 64 + q * 8, 8)], sem_s
        )
    writes[0].wait()
    writes[1].wait()


def kernel(x, weights, indices, W1, W2):
    iflat = indices.reshape(-1).astype(jnp.int32)
    wflat = weights.reshape(-1)

    mesh = plsc.VectorSubcoreMesh(core_axis_name="c", subcore_axis_name="s")
    sc_params = pltpu.CompilerParams(needs_layout_passes=False)

    route = functools.partial(
        pl.kernel,
        mesh=mesh,
        out_type=[
            jax.ShapeDtypeStruct((NTILES, 4, 32), jnp.int32),
            jax.ShapeDtypeStruct((48,), jnp.int32),
            jax.ShapeDtypeStruct((NB * B, D), jnp.float32),
        ],
        scratch_types=[
            pltpu.VMEM((A,), jnp.int32),
            pltpu.VMEM((4, 32), jnp.int32),
            pltpu.VMEM((4, 32), jnp.int32),
            pltpu.VMEM((48,), jnp.int32),
            pltpu.VMEM((32, D), jnp.float32),
            pltpu.VMEM((32, D), jnp.float32),
            pltpu.VMEM((32, D), jnp.float32),
            pltpu.SemaphoreType.DMA,
            pltpu.SemaphoreType.DMA,
        ],
        compiler_params=sc_params,
    )(_route_body)
    pos, gid, xs = route(iflat, x)

    out_sorted = pl.pallas_call(
        _group_body,
        grid_spec=pltpu.PrefetchScalarGridSpec(
            num_scalar_prefetch=1,
            grid=(NB,),
            in_specs=[
                pl.BlockSpec((B, D), lambda b, g: (b, 0)),
                pl.BlockSpec((1, H, D), lambda b, g: (g[b], 0, 0)),
                pl.BlockSpec((1, H, D), lambda b, g: (g[b], 1, 0)),
                pl.BlockSpec((1, D, H), lambda b, g: (g[b], 0, 0)),
            ],
            out_specs=pl.BlockSpec((B, D), lambda b, g: (b, 0)),
        ),
        out_shape=jax.ShapeDtypeStruct((NB * B, D), jnp.float32),
    )(gid, xs, W1, W1, W2)

    combine = functools.partial(
        pl.kernel,
        mesh=mesh,
        out_type=jax.ShapeDtypeStruct((T, D), jnp.float32),
        scratch_types=[
            pltpu.VMEM((4, 32), jnp.int32),
            pltpu.VMEM((CHUNK,), jnp.float32),
            pltpu.VMEM((16, D), jnp.float32),
            pltpu.VMEM((16, D), jnp.float32),
            pltpu.VMEM((16, D), jnp.float32),
            pltpu.VMEM((16, D), jnp.float32),
            pltpu.VMEM((8, D), jnp.float32),
            pltpu.VMEM((8, D), jnp.float32),
            pltpu.SemaphoreType.DMA,
            pltpu.SemaphoreType.DMA,
        ],
        compiler_params=sc_params,
    )(_combine_body)
    return combine(out_sorted, pos, wflat)


# speedup vs baseline: 1.1782x; 1.1782x over previous
import functools

import jax
import jax.numpy as jnp
from jax import lax
from jax.experimental import pallas as pl
from jax.experimental.pallas import tpu as pltpu
from jax.experimental.pallas import tpu_sc as plsc

E = 8
TOPK = 2
D = 1024
H = 512
T = 2048
A = T * TOPK

B = 256
LOGB = 8
NB = A // B + E

NTILES = 32
CHUNK = A // NTILES
NV = CHUNK // 16


def _splat(vec, lanes, e):
    return jnp.zeros((16,), jnp.int32) + jnp.sum(jnp.where(lanes == e, vec, 0))


def _route_body(ifl_hbm, x_hbm, pos_hbm, gid_hbm, xs_hbm, ifl_v, pos_v, tok_v,
                gid_v, rows_a, rows_b, rows_c, sem_g, sem_s):
    c = lax.axis_index("c")
    s = lax.axis_index("s")
    wid = c * 16 + s
    lanes = jnp.arange(16, dtype=jnp.int32)
    bufs = [rows_a, rows_b, rows_c]

    for q in range(4):
        for h in range(2):
            r = wid * CHUNK + q * 32 + h * 16 + lanes
            tok_v[q, pl.ds(h * 16, 16)] = r >> 1
    gathers = []
    for q in range(3):
        gathers.append(
            pltpu.async_copy(x_hbm.at[tok_v.at[q]], bufs[q], sem_g)
        )

    pltpu.sync_copy(ifl_hbm, ifl_v)

    def step(v, carry):
        tot, pre = carry
        kv = ifl_v[pl.ds(v * 16, 16)]
        for e in range(E):
            m = kv == e
            cnt = jnp.sum(jnp.where(m, 1, 0))
            hit = jnp.where(lanes == e, cnt, 0)
            tot = tot + hit
            pre = pre + jnp.where(v < wid * NV, hit, jnp.zeros((16,), jnp.int32))
        return tot, pre

    zero16 = jnp.zeros((16,), jnp.int32)
    tot, pre = lax.fori_loop(0, A // 16, step, (zero16, zero16))

    blocks = (tot + (B - 1)) >> LOGB
    endblk = jnp.cumsum(blocks)
    startblk = endblk - blocks
    base = (startblk << LOGB) + pre

    @pl.when(wid == 0)
    def _():
        for j in range(3):
            bvec = lanes + 16 * j
            gid = jnp.zeros((16,), jnp.int32)
            for e in range(E):
                gid = gid + jnp.where(_splat(endblk, lanes, e) <= bvec, 1, 0)
            gid_v[pl.ds(j * 16, 16)] = jnp.minimum(gid, E - 1)
        pltpu.sync_copy(gid_v, gid_hbm)

    run = base
    for v in range(NV):
        kv = ifl_v[pl.ds((wid * NV + v) * 16, 16)]
        posv = jnp.zeros((16,), jnp.int32)
        for e in range(E):
            m = kv == e
            rank = jnp.cumsum(jnp.where(m, 1, 0)) - 1
            posv = jnp.where(m, _splat(run, lanes, e) + rank, posv)
            cnt = jnp.sum(jnp.where(m, 1, 0))
            run = run + jnp.where(lanes == e, cnt, 0)
        pos_v[v // 2, pl.ds((v % 2) * 16, 16)] = posv

    pltpu.sync_copy(pos_v, pos_hbm.at[wid])

    gathers[0].wait()
    s0 = pltpu.async_copy(bufs[0], xs_hbm.at[pos_v.at[0]], sem_s)
    gathers[1].wait()
    s1 = pltpu.async_copy(bufs[1], xs_hbm.at[pos_v.at[1]], sem_s)
    s0.wait()
    g3 = pltpu.async_copy(x_hbm.at[tok_v.at[3]], bufs[0], sem_g)
    gathers[2].wait()
    s2 = pltpu.async_copy(bufs[2], xs_hbm.at[pos_v.at[2]], sem_s)
    g3.wait()
    s3 = pltpu.async_copy(bufs[0], xs_hbm.at[pos_v.at[3]], sem_s)
    s1.wait()
    s2.wait()
    s3.wait()


def _group_body(gid_ref, x_ref, w1g_ref, w1u_ref, w2_ref, o_ref):
    xb = x_ref[...]
    gate = lax.dot_general(
        xb, w1g_ref[0], (((1,), (1,)), ((), ())),
        preferred_element_type=jnp.float32,
    )
    up = lax.dot_general(
        xb, w1u_ref[0], (((1,), (1,)), ((), ())),
        preferred_element_type=jnp.float32,
    )
    a = gate * lax.logistic(gate) * up
    o_ref[...] = lax.dot_general(
        a, w2_ref[0], (((1,), (1,)), ((), ())),
        preferred_element_type=jnp.float32,
    )


def _combine_body(os_hbm, pos_hbm, w_hbm, y_hbm, pos_v, w_v, r0, r1, r2, r3,
                  y_a, y_b, sem_g, sem_s):
    c = lax.axis_index("c")
    s = lax.axis_index("s")
    wid = c * 16 + s
    rows = [r0, r1, r2, r3]
    ys = [y_a, y_b]

    pltpu.sync_copy(pos_hbm.at[wid], pos_v)
    pltpu.sync_copy(w_hbm.at[pl.ds(wid * CHUNK, CHUNK)], w_v)

    def idx(q):
        return pos_v.at[q // 2, pl.ds((q % 2) * 16, 16)]

    gathers = [pltpu.async_copy(os_hbm.at[idx(q)], rows[q], sem_g)
               for q in range(4)]
    writes = [None, None]
    for q in range(8):
        gathers[q].wait()
        rv = rows[q % 4]
        yv = ys[q % 2]
        if writes[q % 2] is not None:
            writes[q % 2].wait()
        wv = w_v[pl.ds(q * 16, 16)]
        for j in range(8):
            w0 = wv[2 * j]
            w1 = wv[2 * j + 1]

            def inner(u, _, j=j, w0=w0, w1=w1, rv=rv, yv=yv):
                for p in range(8):
                    sl = pl.ds((u * 8 + p) * 16, 16)
                    yv[j, sl] = rv[2 * j, sl] * w0 + rv[2 * j + 1, sl] * w1
                return 0

            lax.fori_loop(0, D // 128, inner, 0)
        if q + 4 < 8:
            gathers.append(
                pltpu.async_copy(os_hbm.at[idx(q + 4)], rv, sem_g)
            )
        writes[q % 2] = pltpu.async_copy(
            yv, y_hbm.at[pl.ds(wid * 64 + q * 8, 8)], sem_s
        )
    writes[0].wait()
    writes[1].wait()


def kernel(x, weights, indices, W1, W2):
    iflat = indices.reshape(-1).astype(jnp.int32)
    wflat = weights.reshape(-1)

    mesh = plsc.VectorSubcoreMesh(core_axis_name="c", subcore_axis_name="s")
    sc_params = pltpu.CompilerParams(needs_layout_passes=False)

    route = functools.partial(
        pl.kernel,
        mesh=mesh,
        out_type=[
            jax.ShapeDtypeStruct((NTILES, 4, 32), jnp.int32),
            jax.ShapeDtypeStruct((48,), jnp.int32),
            jax.ShapeDtypeStruct((NB * B, D), jnp.float32),
        ],
        scratch_types=[
            pltpu.VMEM((A,), jnp.int32),
            pltpu.VMEM((4, 32), jnp.int32),
            pltpu.VMEM((4, 32), jnp.int32),
            pltpu.VMEM((48,), jnp.int32),
            pltpu.VMEM((32, D), jnp.float32),
            pltpu.VMEM((32, D), jnp.float32),
            pltpu.VMEM((32, D), jnp.float32),
            pltpu.SemaphoreType.DMA,
            pltpu.SemaphoreType.DMA,
        ],
        compiler_params=sc_params,
    )(_route_body)
    pos, gid, xs = route(iflat, x)

    out_sorted = pl.pallas_call(
        _group_body,
        grid_spec=pltpu.PrefetchScalarGridSpec(
            num_scalar_prefetch=1,
            grid=(NB,),
            in_specs=[
                pl.BlockSpec((B, D), lambda b, g: (b, 0)),
                pl.BlockSpec((1, H, D), lambda b, g: (g[b], 0, 0)),
                pl.BlockSpec((1, H, D), lambda b, g: (g[b], 1, 0)),
                pl.BlockSpec((1, D, H), lambda b, g: (g[b], 0, 0)),
            ],
            out_specs=pl.BlockSpec((B, D), lambda b, g: (b, 0)),
        ),
        out_shape=jax.ShapeDtypeStruct((NB * B, D), jnp.float32),
    )(gid, xs, W1, W1, W2)

    combine = functools.partial(
        pl.kernel,
        mesh=mesh,
        out_type=jax.ShapeDtypeStruct((T, D), jnp.float32),
        scratch_types=[
            pltpu.VMEM((4, 32), jnp.int32),
            pltpu.VMEM((CHUNK,), jnp.float32),
            pltpu.VMEM((16, D), jnp.float32),
            pltpu.VMEM((16, D), jnp.float32),
            pltpu.VMEM((16, D), jnp.float32),
            pltpu.VMEM((16, D), jnp.float32),
            pltpu.VMEM((8, D), jnp.float32),
            pltpu.VMEM((8, D), jnp.float32),
            pltpu.SemaphoreType.DMA,
            pltpu.SemaphoreType.DMA,
        ],
        compiler_params=sc_params,
    )(_combine_body)
    return combine(out_sorted, pos, wflat)


# speedup vs baseline: 1.1833x; 1.0043x over previous
import functools

import jax
import jax.numpy as jnp
from jax import lax
from jax.experimental import pallas as pl
from jax.experimental.pallas import tpu as pltpu
from jax.experimental.pallas import tpu_sc as plsc

E = 8
TOPK = 2
D = 1024
H = 512
T = 2048
A = T * TOPK

B = 256
LOGB = 8
NB = A // B + E

NTILES = 32
CHUNK = A // NTILES
NV = CHUNK // 16


def _splat(vec, lanes, e):
    return jnp.zeros((16,), jnp.int32) + jnp.sum(jnp.where(lanes == e, vec, 0))


def _route_body(ifl_hbm, x_hbm, pos_hbm, gid_hbm, xs_hbm, ifl_v, pos_v, tok_v,
                gid_v, rows_a, rows_b, rows_c, sem_g, sem_s):
    c = lax.axis_index("c")
    s = lax.axis_index("s")
    wid = c * 16 + s
    lanes = jnp.arange(16, dtype=jnp.int32)
    bufs = [rows_a, rows_b, rows_c]

    for q in range(4):
        for h in range(2):
            r = wid * CHUNK + q * 32 + h * 16 + lanes
            tok_v[q, pl.ds(h * 16, 16)] = r >> 1
    gathers = []
    for q in range(3):
        gathers.append(
            pltpu.async_copy(x_hbm.at[tok_v.at[q]], bufs[q], sem_g)
        )

    pltpu.sync_copy(ifl_hbm, ifl_v)

    zero16 = jnp.zeros((16,), jnp.int32)

    def step(v, accs):
        kv = ifl_v[pl.ds(v * 16, 16)]
        return tuple(
            accs[e] + jnp.where(kv == e, 1, 0) for e in range(E)
        )

    acc_lo = lax.fori_loop(0, wid * NV, step, (zero16,) * E)
    acc_all = lax.fori_loop(wid * NV, A // 16, step, acc_lo)

    tot = zero16
    pre = zero16
    for e in range(E):
        lane_e = lanes == e
        pre = pre + jnp.where(lane_e, jnp.sum(acc_lo[e]), 0)
        tot = tot + jnp.where(lane_e, jnp.sum(acc_all[e]), 0)

    blocks = (tot + (B - 1)) >> LOGB
    endblk = jnp.cumsum(blocks)
    startblk = endblk - blocks
    base = (startblk << LOGB) + pre

    @pl.when(wid == 0)
    def _():
        for j in range(3):
            bvec = lanes + 16 * j
            gid = jnp.zeros((16,), jnp.int32)
            for e in range(E):
                gid = gid + jnp.where(_splat(endblk, lanes, e) <= bvec, 1, 0)
            gid_v[pl.ds(j * 16, 16)] = jnp.minimum(gid, E - 1)
        pltpu.sync_copy(gid_v, gid_hbm)

    run = base
    for v in range(NV):
        kv = ifl_v[pl.ds((wid * NV + v) * 16, 16)]
        posv = jnp.zeros((16,), jnp.int32)
        for e in range(E):
            m = kv == e
            rank = jnp.cumsum(jnp.where(m, 1, 0)) - 1
            posv = jnp.where(m, _splat(run, lanes, e) + rank, posv)
            cnt = jnp.sum(jnp.where(m, 1, 0))
            run = run + jnp.where(lanes == e, cnt, 0)
        pos_v[v // 2, pl.ds((v % 2) * 16, 16)] = posv

    pltpu.sync_copy(pos_v, pos_hbm.at[wid])

    gathers[0].wait()
    s0 = pltpu.async_copy(bufs[0], xs_hbm.at[pos_v.at[0]], sem_s)
    gathers[1].wait()
    s1 = pltpu.async_copy(bufs[1], xs_hbm.at[pos_v.at[1]], sem_s)
    s0.wait()
    g3 = pltpu.async_copy(x_hbm.at[tok_v.at[3]], bufs[0], sem_g)
    gathers[2].wait()
    s2 = pltpu.async_copy(bufs[2], xs_hbm.at[pos_v.at[2]], sem_s)
    g3.wait()
    s3 = pltpu.async_copy(bufs[0], xs_hbm.at[pos_v.at[3]], sem_s)
    s1.wait()
    s2.wait()
    s3.wait()


def _group_body(gid_ref, x_ref, w1g_ref, w1u_ref, w2_ref, o_ref):
    xb = x_ref[...]
    gate = lax.dot_general(
        xb, w1g_ref[0], (((1,), (1,)), ((), ())),
        preferred_element_type=jnp.float32,
    )
    up = lax.dot_general(
        xb, w1u_ref[0], (((1,), (1,)), ((), ())),
        preferred_element_type=jnp.float32,
    )
    a = gate * lax.logistic(gate) * up
    o_ref[...] = lax.dot_general(
        a, w2_ref[0], (((1,), (1,)), ((), ())),
        preferred_element_type=jnp.float32,
    )


def _combine_body(os_hbm, pos_hbm, w_hbm, y_hbm, pos_v, w_v, r0, r1, r2, r3,
                  y_a, y_b, sem_g, sem_s):
    c = lax.axis_index("c")
    s = lax.axis_index("s")
    wid = c * 16 + s
    rows = [r0, r1, r2, r3]
    ys = [y_a, y_b]

    pltpu.sync_copy(pos_hbm.at[wid], pos_v)
    pltpu.sync_copy(w_hbm.at[pl.ds(wid * CHUNK, CHUNK)], w_v)

    def idx(q):
        return pos_v.at[q // 2, pl.ds((q % 2) * 16, 16)]

    gathers = [pltpu.async_copy(os_hbm.at[idx(q)], rows[q], sem_g)
               for q in range(4)]
    writes = [None, None]
    for q in range(8):
        gathers[q].wait()
        rv = rows[q % 4]
        yv = ys[q % 2]
        if writes[q % 2] is not None:
            writes[q % 2].wait()
        wv = w_v[pl.ds(q * 16, 16)]
        for j in range(8):
            w0 = wv[2 * j]
            w1 = wv[2 * j + 1]

            def inner(u, _, j=j, w0=w0, w1=w1, rv=rv, yv=yv):
                for p in range(8):
                    sl = pl.ds((u * 8 + p) * 16, 16)
                    yv[j, sl] = rv[2 * j, sl] * w0 + rv[2 * j + 1, sl] * w1
                return 0

            lax.fori_loop(0, D // 128, inner, 0)
        if q + 4 < 8:
            gathers.append(
                pltpu.async_copy(os_hbm.at[idx(q + 4)], rv, sem_g)
            )
        writes[q % 2] = pltpu.async_copy(
            yv, y_hbm.at[pl.ds(wid * 64 + q * 8, 8)], sem_s
        )
    writes[0].wait()
    writes[1].wait()


def kernel(x, weights, indices, W1, W2):
    iflat = indices.reshape(-1).astype(jnp.int32)
    wflat = weights.reshape(-1)

    mesh = plsc.VectorSubcoreMesh(core_axis_name="c", subcore_axis_name="s")
    sc_params = pltpu.CompilerParams(needs_layout_passes=False)

    route = functools.partial(
        pl.kernel,
        mesh=mesh,
        out_type=[
            jax.ShapeDtypeStruct((NTILES, 4, 32), jnp.int32),
            jax.ShapeDtypeStruct((48,), jnp.int32),
            jax.ShapeDtypeStruct((NB * B, D), jnp.float32),
        ],
        scratch_types=[
            pltpu.VMEM((A,), jnp.int32),
            pltpu.VMEM((4, 32), jnp.int32),
            pltpu.VMEM((4, 32), jnp.int32),
            pltpu.VMEM((48,), jnp.int32),
            pltpu.VMEM((32, D), jnp.float32),
            pltpu.VMEM((32, D), jnp.float32),
            pltpu.VMEM((32, D), jnp.float32),
            pltpu.SemaphoreType.DMA,
            pltpu.SemaphoreType.DMA,
        ],
        compiler_params=sc_params,
    )(_route_body)
    pos, gid, xs = route(iflat, x)

    out_sorted = pl.pallas_call(
        _group_body,
        grid_spec=pltpu.PrefetchScalarGridSpec(
            num_scalar_prefetch=1,
            grid=(NB,),
            in_specs=[
                pl.BlockSpec((B, D), lambda b, g: (b, 0)),
                pl.BlockSpec((1, H, D), lambda b, g: (g[b], 0, 0)),
                pl.BlockSpec((1, H, D), lambda b, g: (g[b], 1, 0)),
                pl.BlockSpec((1, D, H), lambda b, g: (g[b], 0, 0)),
            ],
            out_specs=pl.BlockSpec((B, D), lambda b, g: (b, 0)),
        ),
        out_shape=jax.ShapeDtypeStruct((NB * B, D), jnp.float32),
    )(gid, xs, W1, W1, W2)

    combine = functools.partial(
        pl.kernel,
        mesh=mesh,
        out_type=jax.ShapeDtypeStruct((T, D), jnp.float32),
        scratch_types=[
            pltpu.VMEM((4, 32), jnp.int32),
            pltpu.VMEM((CHUNK,), jnp.float32),
            pltpu.VMEM((16, D), jnp.float32),
            pltpu.VMEM((16, D), jnp.float32),
            pltpu.VMEM((16, D), jnp.float32),
            pltpu.VMEM((16, D), jnp.float32),
            pltpu.VMEM((8, D), jnp.float32),
            pltpu.VMEM((8, D), jnp.float32),
            pltpu.SemaphoreType.DMA,
            pltpu.SemaphoreType.DMA,
        ],
        compiler_params=sc_params,
    )(_combine_body)
    return combine(out_sorted, pos, wflat)
